# baseline (speedup 1.0000x reference)
import functools

import jax
import jax.numpy as jnp
from jax import lax
from jax.experimental import pallas as pl
from jax.experimental.pallas import tpu as pltpu
from jax.experimental.pallas import tpu_sc as plsc

NC = 2
NS = 16
NW = NC * NS
NB = 4
DP = 128
TP = 72


@jax.jit
def _embed_lookup(table, idx):
    R, S = idx.shape
    V, _ = table.shape
    RW = R // NW
    nstep = RW
    chunks = [(0, 128), (128, S - 128)] if S > 128 else [(0, S)]
    mesh = plsc.VectorSubcoreMesh(
        core_axis_name="c", subcore_axis_name="s", num_cores=NC, num_subcores=NS
    )

    @functools.partial(
        pl.kernel,
        out_type=jax.ShapeDtypeStruct((R, S, DP), jnp.float32),
        mesh=mesh,
        scratch_types=[
            pltpu.VMEM((RW, S), jnp.int32),
            pltpu.VMEM((NB, S, TP), jnp.float32),
            pltpu.SemaphoreType.DMA,
            pltpu.SemaphoreType.DMA,
        ],
        compiler_params=pltpu.CompilerParams(use_tc_tiling_on_sc=False),
    )
    def body(table_hbm, idx_hbm, out_hbm, idx_v, rows_v, gsem, osem):
        wid = lax.axis_index("s") * NC + lax.axis_index("c")
        row0 = wid * RW
        pltpu.sync_copy(idx_hbm.at[pl.ds(row0, RW)], idx_v)

        def gather_descs(b, r, make):
            return [
                make(
                    table_hbm.at[idx_v.at[r, pl.ds(o, w)]],
                    rows_v.at[b, pl.ds(o, w)],
                    gsem,
                )
                for (o, w) in chunks
            ]

        def fire(b, r):
            gather_descs(b, r, pltpu.async_copy)

        def wait_gathers(b, r):
            for d in gather_descs(b, r, pltpu.make_async_copy):
                d.wait()

        def start_out(b, r):
            pltpu.async_copy(
                rows_v.at[b, :, pl.ds(0, 64)],
                out_hbm.at[row0 + r, :, pl.ds(0, 64)],
                osem,
            )

        def wait_out(b, r):
            pltpu.make_async_copy(
                rows_v.at[b, :, pl.ds(0, 64)],
                out_hbm.at[row0 + r, :, pl.ds(0, 64)],
                osem,
            ).wait()

        for s in range(NB - 1):
            fire(s, s)

        def step_fn(s, carry):
            b = lax.rem(s, NB)
            wait_gathers(b, s)
            start_out(b, s)
            s2 = s + (NB - 1)
            b2 = lax.rem(s2, NB)

            @pl.when(s2 < nstep)
            def _():
                @pl.when(s >= 1)
                def _():
                    wait_out(b2, s - 1)

                fire(b2, s2)

            return carry

        lax.fori_loop(0, nstep, step_fn, 0)
        for s in range(nstep - NB, nstep):
            wait_out(s % NB, s)

    return body(table, idx)


def kernel(input, table):
    tablep = jnp.pad(table, ((0, 0), (0, TP - table.shape[1])))
    out128 = _embed_lookup(tablep, input.astype(jnp.int32))
    return out128[:, :, : table.shape[1]]


# speedup vs baseline: 1.5205x; 1.5205x over previous
import functools

import jax
import jax.numpy as jnp
from jax import lax
from jax.experimental import pallas as pl
from jax.experimental.pallas import tpu as pltpu
from jax.experimental.pallas import tpu_sc as plsc

NC = 2
NS = 16
NW = NC * NS
NB = 4
DP = 128
TP = 128


@jax.jit
def _embed_lookup(table, idx):
    R, S = idx.shape
    V, _ = table.shape
    RW = R // NW
    nstep = RW
    chunks = [(0, 128), (128, S - 128)] if S > 128 else [(0, S)]
    mesh = plsc.VectorSubcoreMesh(
        core_axis_name="c", subcore_axis_name="s", num_cores=NC, num_subcores=NS
    )

    @functools.partial(
        pl.kernel,
        out_type=jax.ShapeDtypeStruct((R, S, DP), jnp.float32),
        mesh=mesh,
        scratch_types=[
            pltpu.VMEM((RW, S), jnp.int32),
            pltpu.VMEM((NB, S, TP), jnp.float32),
            pltpu.SemaphoreType.DMA,
            pltpu.SemaphoreType.DMA,
        ],
        compiler_params=pltpu.CompilerParams(use_tc_tiling_on_sc=False),
    )
    def body(table_hbm, idx_hbm, out_hbm, idx_v, rows_v, gsem, osem):
        wid = lax.axis_index("s") * NC + lax.axis_index("c")
        row0 = wid * RW
        pltpu.sync_copy(idx_hbm.at[pl.ds(row0, RW)], idx_v)

        def gather_descs(b, r, make):
            return [
                make(
                    table_hbm.at[idx_v.at[r, pl.ds(o, w)]],
                    rows_v.at[b, pl.ds(o, w)],
                    gsem,
                )
                for (o, w) in chunks
            ]

        def fire(b, r):
            gather_descs(b, r, pltpu.async_copy)

        def wait_gathers(b, r):
            for d in gather_descs(b, r, pltpu.make_async_copy):
                d.wait()

        def start_out(b, r):
            pltpu.async_copy(
                rows_v.at[b, :, pl.ds(0, 64)],
                out_hbm.at[row0 + r, :, pl.ds(0, 64)],
                osem,
            )

        def wait_out(b, r):
            pltpu.make_async_copy(
                rows_v.at[b, :, pl.ds(0, 64)],
                out_hbm.at[row0 + r, :, pl.ds(0, 64)],
                osem,
            ).wait()

        for s in range(NB - 1):
            fire(s, s)

        def step_fn(s, carry):
            b = lax.rem(s, NB)
            wait_gathers(b, s)
            start_out(b, s)
            s2 = s + (NB - 1)
            b2 = lax.rem(s2, NB)

            @pl.when(s2 < nstep)
            def _():
                @pl.when(s >= 1)
                def _():
                    wait_out(b2, s - 1)

                fire(b2, s2)

            return carry

        lax.fori_loop(0, nstep, step_fn, 0)
        for s in range(nstep - NB, nstep):
            wait_out(s % NB, s)

    return body(table, idx)


def kernel(input, table):
    tablep = jnp.pad(table, ((0, 0), (0, TP - table.shape[1])))
    out128 = _embed_lookup(tablep, input.astype(jnp.int32))
    return out128[:, :, : table.shape[1]]


# speedup vs baseline: 1.7312x; 1.1385x over previous
import functools

import jax
import jax.numpy as jnp
from jax import lax
from jax.experimental import pallas as pl
from jax.experimental.pallas import tpu as pltpu
from jax.experimental.pallas import tpu_sc as plsc

NC = 2
NS = 16
NW = NC * NS
NB = 4
DP = 128
TP = 128


@jax.jit
def _embed_lookup(table, idx):
    R, S = idx.shape
    V, _ = table.shape
    RW = R // NW
    nstep = RW
    chunks = [(0, 128), (128, S - 128)] if S > 128 else [(0, S)]
    mesh = plsc.VectorSubcoreMesh(
        core_axis_name="c", subcore_axis_name="s", num_cores=NC, num_subcores=NS
    )

    @functools.partial(
        pl.kernel,
        out_type=jax.ShapeDtypeStruct((R, S, DP), jnp.float32),
        mesh=mesh,
        scratch_types=[
            pltpu.VMEM((RW, S), jnp.int32),
            pltpu.VMEM((NB, S, TP), jnp.float32),
            pltpu.SemaphoreType.DMA,
            pltpu.SemaphoreType.DMA,
        ],
        compiler_params=pltpu.CompilerParams(use_tc_tiling_on_sc=False),
    )
    def body(table_hbm, idx_hbm, out_hbm, idx_v, rows_v, gsem, osem):
        wid = lax.axis_index("s") * NC + lax.axis_index("c")
        row0 = wid * RW
        pltpu.sync_copy(idx_hbm.at[pl.ds(row0, RW)], idx_v)

        def gather_descs(b, r, make):
            return [
                make(
                    table_hbm.at[idx_v.at[r, pl.ds(o, w)]],
                    rows_v.at[b, pl.ds(o, w)],
                    gsem,
                )
                for (o, w) in chunks
            ]

        def fire(b, r):
            gather_descs(b, r, pltpu.async_copy)

        def wait_gathers(b, r):
            for d in gather_descs(b, r, pltpu.make_async_copy):
                d.wait()

        def start_out(b, r):
            pltpu.async_copy(
                rows_v.at[b, :, pl.ds(0, 64)],
                out_hbm.at[row0 + r, :, pl.ds(0, 64)],
                osem,
            )

        def wait_out(b, r):
            pltpu.make_async_copy(
                rows_v.at[b, :, pl.ds(0, 64)],
                out_hbm.at[row0 + r, :, pl.ds(0, 64)],
                osem,
            ).wait()

        for s in range(NB - 1):
            fire(s, s)

        def step_fn(s, carry):
            b = lax.rem(s, NB)
            wait_gathers(b, s)
            start_out(b, s)
            s2 = s + (NB - 1)
            b2 = lax.rem(s2, NB)

            @pl.when(s2 < nstep)
            def _():
                @pl.when(s >= 1)
                def _():
                    wait_out(b2, s - 1)

                fire(b2, s2)

            return carry

        lax.fori_loop(0, nstep, step_fn, 0)
        for s in range(nstep - NB, nstep):
            wait_out(s % NB, s)

    return body(table, idx)


def kernel(input, table):
    sel = jnp.eye(table.shape[1], TP, dtype=table.dtype)
    tablep = jax.lax.dot_general(
        table, sel, (((1,), (0,)), ((), ())),
        precision=jax.lax.Precision.HIGHEST,
        preferred_element_type=jnp.float32,
    )
    out128 = _embed_lookup(tablep, input.astype(jnp.int32))
    return out128[:, :, : table.shape[1]]
